# baseline (speedup 1.0000x reference)
import functools

import jax
import jax.numpy as jnp
from jax import lax
from jax.experimental import pallas as pl
from jax.experimental.pallas import tpu as pltpu
from jax.experimental.pallas import tpu_sc as plsc

NC = 2
NS = 16
NW = NC * NS
K = 128


def _sc_mesh():
    return plsc.VectorSubcoreMesh(core_axis_name="c", subcore_axis_name="s",
                                  num_cores=NC, num_subcores=NS)


@functools.lru_cache(maxsize=None)
def _make_sc_degree(n_pad, cpt, width):
    rpt = n_pad // NS

    @functools.partial(
        pl.kernel,
        out_type=jax.ShapeDtypeStruct((NC, n_pad, width), jnp.float32),
        mesh=_sc_mesh(),
        scratch_types=[
            pltpu.VMEM((cpt, K), jnp.int32),
            pltpu.VMEM((K, width), jnp.float32),
            pltpu.VMEM_SHARED((n_pad, width), jnp.float32),
        ],
    )
    def deg_kernel(dst_hbm, zeros_hbm, ones_hbm, out_hbm, idx_v, ones_v, acc):
        c = lax.axis_index("c")
        s = lax.axis_index("s")
        wid = c * NS + s
        pltpu.sync_copy(zeros_hbm.at[pl.ds(s * rpt, rpt)],
                        acc.at[pl.ds(s * rpt, rpt)])
        pltpu.sync_copy(ones_hbm, ones_v)
        pltpu.sync_copy(dst_hbm.at[pl.ds(wid * cpt, cpt)], idx_v)
        plsc.subcore_barrier()

        def body(j, carry):
            pltpu.sync_copy(ones_v, acc.at[idx_v.at[j]], add=True)
            return carry

        lax.fori_loop(0, cpt, body, 0)
        plsc.subcore_barrier()
        pltpu.sync_copy(acc.at[pl.ds(s * rpt, rpt)],
                        out_hbm.at[c, pl.ds(s * rpt, rpt)])

    return deg_kernel


@functools.lru_cache(maxsize=None)
def _make_sc_agg(n_pad, ct, width, q0):
    rpt = n_pad // NS
    q1 = (ct - NS * q0) // NS
    PCMAX = 48

    def phase_list(q):
        ph = [PCMAX] * (q // PCMAX)
        if q % PCMAX:
            ph.append(q % PCMAX)
        return ph

    @functools.partial(
        pl.kernel,
        out_type=jax.ShapeDtypeStruct((NC, n_pad, width), jnp.float32),
        mesh=_sc_mesh(),
        scratch_types=[
            pltpu.VMEM((PCMAX, K), jnp.int32),
            pltpu.VMEM((PCMAX, K), jnp.int32),
            pltpu.VMEM((K, width), jnp.float32),
            pltpu.VMEM((K, width), jnp.float32),
            pltpu.VMEM_SHARED((n_pad, width), jnp.float32),
            pltpu.SemaphoreType.DMA,
        ],
    )
    def agg_kernel(y_hbm, src_hbm, dst_hbm, zeros_hbm, out_hbm,
                   src_v, dst_v, rows0, rows1, acc, sem):
        c = lax.axis_index("c")
        s = lax.axis_index("s")
        pltpu.sync_copy(zeros_hbm.at[pl.ds(s * rpt, rpt)],
                        acc.at[pl.ds(s * rpt, rpt)])
        plsc.subcore_barrier()

        def run_phases(base, phases):
            off = 0
            for PC in phases:
                pltpu.sync_copy(src_hbm.at[pl.ds(base + off, PC)],
                                src_v.at[pl.ds(0, PC)])
                pltpu.sync_copy(dst_hbm.at[pl.ds(base + off, PC)],
                                dst_v.at[pl.ds(0, PC)])
                off += PC

                def gather(j, buf):
                    pltpu.async_copy(
                        y_hbm.at[src_v.at[j, pl.ds(0, K // 2)]],
                        buf.at[pl.ds(0, K // 2)], sem)
                    pltpu.async_copy(
                        y_hbm.at[src_v.at[j, pl.ds(K // 2, K // 2)]],
                        buf.at[pl.ds(K // 2, K // 2)], sem)

                def gather_wait(j, buf):
                    pltpu.make_async_copy(
                        y_hbm.at[src_v.at[j, pl.ds(0, K // 2)]],
                        buf.at[pl.ds(0, K // 2)], sem).wait()
                    pltpu.make_async_copy(
                        y_hbm.at[src_v.at[j, pl.ds(0, K // 2)]],
                        buf.at[pl.ds(K // 2, K // 2)], sem).wait()

                gather(0, rows0)

                def body(i, carry):
                    j0 = 2 * i
                    gather_wait(j0, rows0)
                    gather(j0 + 1, rows1)
                    pltpu.sync_copy(rows0, acc.at[dst_v.at[j0]], add=True)
                    gather_wait(j0 + 1, rows1)

                    @pl.when(j0 + 2 < PC)
                    def _():
                        gather(j0 + 2, rows0)

                    pltpu.sync_copy(rows1, acc.at[dst_v.at[j0 + 1]], add=True)
                    return carry

                lax.fori_loop(0, PC // 2, body, 0)

        @pl.when(c == 0)
        def _():
            run_phases(s * q0, phase_list(q0))

        if q1:
            @pl.when(c == 1)
            def _():
                run_phases(NS * q0 + s * q1, phase_list(q1))

        plsc.subcore_barrier()
        pltpu.sync_copy(acc.at[pl.ds(s * rpt, rpt)],
                        out_hbm.at[c, pl.ds(s * rpt, rpt)])

    return agg_kernel


def _dinv_col(degp_ref, n, n_pad):
    deg = degp_ref[0, :, 0:1] + degp_ref[1, :, 0:1] + 1.0
    mask = lax.broadcasted_iota(jnp.int32, (n_pad, 1), 0) < n
    return jnp.where(mask, lax.rsqrt(deg), 0.0), mask


def _tc_first(x_pad, w, degp, n):
    n_pad = x_pad.shape[0]

    def body(degp_ref, x_ref, w_ref, o_ref):
        dinv, _ = _dinv_col(degp_ref, n, n_pad)
        o_ref[...] = dinv * jnp.dot(x_ref[...], w_ref[...],
                                    preferred_element_type=jnp.float32)

    return pl.pallas_call(
        body,
        out_shape=jax.ShapeDtypeStruct((n_pad, w.shape[1]), jnp.float32),
    )(degp, x_pad, w)


def _bn_relu(aggp_ref, y_ref, degp_ref, b_ref, g_ref, be_ref, n, n_pad):
    dinv, mask = _dinv_col(degp_ref, n, n_pad)
    h = dinv * (aggp_ref[0] + aggp_ref[1] + y_ref[...]) + b_ref[...]
    h = jnp.where(mask, h, 0.0)
    m = jnp.sum(h, axis=0, keepdims=True) * (1.0 / n)
    ctr = jnp.where(mask, h - m, 0.0)
    v = jnp.sum(ctr * ctr, axis=0, keepdims=True) * (1.0 / n)
    hn = g_ref[...] * ctr * lax.rsqrt(v + 1e-5) + be_ref[...]
    hr = jnp.maximum(hn, 0.0)
    return jnp.where(mask, hr, 0.0), dinv


def _tc_mid(aggp, y, degp, b, g, be, w, n):
    n_pad = y.shape[0]

    def body(aggp_ref, y_ref, degp_ref, b_ref, g_ref, be_ref, w_ref, o_ref):
        hr, dinv = _bn_relu(aggp_ref, y_ref, degp_ref, b_ref, g_ref, be_ref,
                            n, n_pad)
        o_ref[...] = dinv * jnp.dot(hr, w_ref[...],
                                    preferred_element_type=jnp.float32)

    return pl.pallas_call(
        body,
        out_shape=jax.ShapeDtypeStruct((n_pad, w.shape[1]), jnp.float32),
    )(aggp, y, degp, b, g, be, w)


def _tc_mid_nodot(aggp, y, degp, b, g, be, n):
    n_pad = y.shape[0]

    def body(aggp_ref, y_ref, degp_ref, b_ref, g_ref, be_ref, o_ref):
        hr, dinv = _bn_relu(aggp_ref, y_ref, degp_ref, b_ref, g_ref, be_ref,
                            n, n_pad)
        o_ref[...] = dinv * hr

    return pl.pallas_call(
        body,
        out_shape=jax.ShapeDtypeStruct((n_pad, y.shape[1]), jnp.float32),
    )(aggp, y, degp, b, g, be)


def _tc_last(aggp, z, degp, w3, b3, n):
    n_pad = z.shape[0]

    def body(aggp_ref, z_ref, degp_ref, w3_ref, b3_ref, o_ref):
        dinv, _ = _dinv_col(degp_ref, n, n_pad)
        t = dinv * (aggp_ref[0] + aggp_ref[1] + z_ref[...])
        logits = jnp.dot(t, w3_ref[...],
                         preferred_element_type=jnp.float32) + b3_ref[...]
        mx = jnp.max(logits, axis=-1, keepdims=True)
        sh = logits - mx
        lse = jnp.log(jnp.sum(jnp.exp(sh), axis=-1, keepdims=True))
        o_ref[...] = sh - lse

    return pl.pallas_call(
        body,
        out_shape=jax.ShapeDtypeStruct((n_pad, w3.shape[1]), jnp.float32),
    )(aggp, z, degp, w3, b3)


def kernel(x, adj_t, W1, b1, g1, be1, W2, b2, g2, be2, W3, b3):
    n, d = x.shape
    e = adj_t.shape[1]
    cpt = 8 * (-(-e // (NW * K * 8)))
    e_pad = NW * K * cpt
    rpt = 8 * (-(-(n + 1) // (NS * 8)))
    n_pad = NS * rpt

    pad = jnp.full((e_pad - e,), n, dtype=jnp.int32)
    src2d = jnp.concatenate([adj_t[0], pad]).reshape(NW * cpt, K)
    dst2d = jnp.concatenate([adj_t[1], pad]).reshape(NW * cpt, K)
    zeros_w = jnp.zeros((n_pad, d), jnp.float32)
    ones_w = jnp.ones((K, d), jnp.float32)
    x_pad = jnp.concatenate([x, jnp.zeros((n_pad - n, d), x.dtype)])

    degp = _make_sc_degree(n_pad, cpt, d)(dst2d, zeros_w, ones_w)
    ct = NW * cpt
    q_total = ct // NS
    q0 = q_total
    agg = _make_sc_agg(n_pad, ct, d, q0)

    y1 = _tc_first(x_pad, W1, degp, n)
    a1 = agg(y1, src2d, dst2d, zeros_w)
    y2 = _tc_mid(a1, y1, degp, b1, g1, be1, W2, n)
    a2 = agg(y2, src2d, dst2d, zeros_w)
    z = _tc_mid_nodot(a2, y2, degp, b2, g2, be2, n)
    a3 = agg(z, src2d, dst2d, zeros_w)
    out = _tc_last(a3, z, degp, W3, b3, n)
    return out[:n]


# speedup vs baseline: 1.3287x; 1.3287x over previous
import functools

import jax
import jax.numpy as jnp
from jax import lax
from jax.experimental import pallas as pl
from jax.experimental.pallas import tpu as pltpu
from jax.experimental.pallas import tpu_sc as plsc

NC = 2
NS = 16
NW = NC * NS
K = 128


def _sc_mesh():
    return plsc.VectorSubcoreMesh(core_axis_name="c", subcore_axis_name="s",
                                  num_cores=NC, num_subcores=NS)


@functools.lru_cache(maxsize=None)
def _make_sc_degree(n_pad, cpt, width):
    rpt = n_pad // NS

    @functools.partial(
        pl.kernel,
        out_type=jax.ShapeDtypeStruct((NC, n_pad, width), jnp.float32),
        mesh=_sc_mesh(),
        scratch_types=[
            pltpu.VMEM((cpt, K), jnp.int32),
            pltpu.VMEM((K, width), jnp.float32),
            pltpu.VMEM_SHARED((n_pad, width), jnp.float32),
        ],
    )
    def deg_kernel(dst_hbm, zeros_hbm, ones_hbm, out_hbm, idx_v, ones_v, acc):
        c = lax.axis_index("c")
        s = lax.axis_index("s")
        wid = c * NS + s
        pltpu.sync_copy(zeros_hbm.at[pl.ds(s * rpt, rpt)],
                        acc.at[pl.ds(s * rpt, rpt)])
        pltpu.sync_copy(ones_hbm, ones_v)
        pltpu.sync_copy(dst_hbm.at[pl.ds(wid * cpt, cpt)], idx_v)
        plsc.subcore_barrier()

        def body(j, carry):
            pltpu.sync_copy(ones_v, acc.at[idx_v.at[j]], add=True)
            return carry

        lax.fori_loop(0, cpt, body, 0)
        plsc.subcore_barrier()
        pltpu.sync_copy(acc.at[pl.ds(s * rpt, rpt)],
                        out_hbm.at[c, pl.ds(s * rpt, rpt)])

    return deg_kernel


@functools.lru_cache(maxsize=None)
def _make_sc_agg(n_pad, ct, width, q0):
    rpt = n_pad // NS
    q1 = (ct - NS * q0) // NS
    PCMAX = 48

    def phase_list(q):
        ph = [PCMAX] * (q // PCMAX)
        if q % PCMAX:
            ph.append(q % PCMAX)
        return ph

    @functools.partial(
        pl.kernel,
        out_type=jax.ShapeDtypeStruct((NC, n_pad, width), jnp.float32),
        mesh=_sc_mesh(),
        scratch_types=[
            pltpu.VMEM((PCMAX, K), jnp.int32),
            pltpu.VMEM((PCMAX, K), jnp.int32),
            pltpu.VMEM((K, width), jnp.float32),
            pltpu.VMEM((K, width), jnp.float32),
            pltpu.VMEM_SHARED((n_pad, width), jnp.float32),
            pltpu.SemaphoreType.DMA,
        ],
    )
    def agg_kernel(y_hbm, src_hbm, dst_hbm, zeros_hbm, out_hbm,
                   src_v, dst_v, rows0, rows1, acc, sem):
        c = lax.axis_index("c")
        s = lax.axis_index("s")
        pltpu.sync_copy(zeros_hbm.at[pl.ds(s * rpt, rpt)],
                        acc.at[pl.ds(s * rpt, rpt)])
        plsc.subcore_barrier()

        def run_phases(base, phases):
            off = 0
            for PC in phases:
                pltpu.sync_copy(src_hbm.at[pl.ds(base + off, PC)],
                                src_v.at[pl.ds(0, PC)])
                pltpu.sync_copy(dst_hbm.at[pl.ds(base + off, PC)],
                                dst_v.at[pl.ds(0, PC)])
                off += PC

                def gather(j, buf):
                    pltpu.async_copy(
                        y_hbm.at[src_v.at[j, pl.ds(0, K // 2)]],
                        buf.at[pl.ds(0, K // 2)], sem)
                    pltpu.async_copy(
                        y_hbm.at[src_v.at[j, pl.ds(K // 2, K // 2)]],
                        buf.at[pl.ds(K // 2, K // 2)], sem)

                def gather_wait(j, buf):
                    pltpu.make_async_copy(
                        y_hbm.at[src_v.at[j, pl.ds(0, K // 2)]],
                        buf.at[pl.ds(0, K // 2)], sem).wait()
                    pltpu.make_async_copy(
                        y_hbm.at[src_v.at[j, pl.ds(0, K // 2)]],
                        buf.at[pl.ds(K // 2, K // 2)], sem).wait()

                gather(0, rows0)

                def body(i, carry):
                    j0 = 2 * i
                    gather_wait(j0, rows0)
                    gather(j0 + 1, rows1)
                    pltpu.sync_copy(rows0, acc.at[dst_v.at[j0]], add=True)
                    gather_wait(j0 + 1, rows1)

                    @pl.when(j0 + 2 < PC)
                    def _():
                        gather(j0 + 2, rows0)

                    pltpu.sync_copy(rows1, acc.at[dst_v.at[j0 + 1]], add=True)
                    return carry

                lax.fori_loop(0, PC // 2, body, 0)

        @pl.when(c == 0)
        def _():
            run_phases(s * q0, phase_list(q0))

        if q1:
            @pl.when(c == 1)
            def _():
                run_phases(NS * q0 + s * q1, phase_list(q1))

        plsc.subcore_barrier()
        pltpu.sync_copy(acc.at[pl.ds(s * rpt, rpt)],
                        out_hbm.at[c, pl.ds(s * rpt, rpt)])

    return agg_kernel


def _dinv_col(degp_ref, n, n_pad):
    deg = degp_ref[0, :, 0:1] + degp_ref[1, :, 0:1] + 1.0
    mask = lax.broadcasted_iota(jnp.int32, (n_pad, 1), 0) < n
    return jnp.where(mask, lax.rsqrt(deg), 0.0), mask


def _tc_first(x_pad, w, degp, n):
    n_pad = x_pad.shape[0]

    def body(degp_ref, x_ref, w_ref, o_ref):
        dinv, _ = _dinv_col(degp_ref, n, n_pad)
        o_ref[...] = dinv * jnp.dot(x_ref[...], w_ref[...],
                                    preferred_element_type=jnp.float32)

    return pl.pallas_call(
        body,
        out_shape=jax.ShapeDtypeStruct((n_pad, w.shape[1]), jnp.float32),
    )(degp, x_pad, w)


def _bn_relu(aggp_ref, y_ref, degp_ref, b_ref, g_ref, be_ref, n, n_pad):
    dinv, mask = _dinv_col(degp_ref, n, n_pad)
    h = dinv * (aggp_ref[0] + aggp_ref[1] + y_ref[...]) + b_ref[...]
    h = jnp.where(mask, h, 0.0)
    m = jnp.sum(h, axis=0, keepdims=True) * (1.0 / n)
    ctr = jnp.where(mask, h - m, 0.0)
    v = jnp.sum(ctr * ctr, axis=0, keepdims=True) * (1.0 / n)
    hn = g_ref[...] * ctr * lax.rsqrt(v + 1e-5) + be_ref[...]
    hr = jnp.maximum(hn, 0.0)
    return jnp.where(mask, hr, 0.0), dinv


def _tc_mid(aggp, y, degp, b, g, be, w, n):
    n_pad = y.shape[0]

    def body(aggp_ref, y_ref, degp_ref, b_ref, g_ref, be_ref, w_ref, o_ref):
        hr, dinv = _bn_relu(aggp_ref, y_ref, degp_ref, b_ref, g_ref, be_ref,
                            n, n_pad)
        o_ref[...] = dinv * jnp.dot(hr, w_ref[...],
                                    preferred_element_type=jnp.float32)

    return pl.pallas_call(
        body,
        out_shape=jax.ShapeDtypeStruct((n_pad, w.shape[1]), jnp.float32),
    )(aggp, y, degp, b, g, be, w)


def _tc_mid_nodot(aggp, y, degp, b, g, be, n):
    n_pad = y.shape[0]

    def body(aggp_ref, y_ref, degp_ref, b_ref, g_ref, be_ref, o_ref):
        hr, dinv = _bn_relu(aggp_ref, y_ref, degp_ref, b_ref, g_ref, be_ref,
                            n, n_pad)
        o_ref[...] = dinv * hr

    return pl.pallas_call(
        body,
        out_shape=jax.ShapeDtypeStruct((n_pad, y.shape[1]), jnp.float32),
    )(aggp, y, degp, b, g, be)


def _tc_last(aggp, z, degp, w3, b3, n):
    n_pad = z.shape[0]

    def body(aggp_ref, z_ref, degp_ref, w3_ref, b3_ref, o_ref):
        dinv, _ = _dinv_col(degp_ref, n, n_pad)
        t = dinv * (aggp_ref[0] + aggp_ref[1] + z_ref[...])
        logits = jnp.dot(t, w3_ref[...],
                         preferred_element_type=jnp.float32) + b3_ref[...]
        mx = jnp.max(logits, axis=-1, keepdims=True)
        sh = logits - mx
        lse = jnp.log(jnp.sum(jnp.exp(sh), axis=-1, keepdims=True))
        o_ref[...] = sh - lse

    return pl.pallas_call(
        body,
        out_shape=jax.ShapeDtypeStruct((n_pad, w3.shape[1]), jnp.float32),
    )(aggp, z, degp, w3, b3)


def kernel(x, adj_t, W1, b1, g1, be1, W2, b2, g2, be2, W3, b3):
    n, d = x.shape
    e = adj_t.shape[1]
    cpt = 8 * (-(-e // (NW * K * 8)))
    e_pad = NW * K * cpt
    rpt = 8 * (-(-(n + 1) // (NS * 8)))
    n_pad = NS * rpt

    pad = jnp.full((e_pad - e,), n, dtype=jnp.int32)
    src2d = jnp.concatenate([adj_t[0], pad]).reshape(NW * cpt, K)
    dst2d = jnp.concatenate([adj_t[1], pad]).reshape(NW * cpt, K)
    zeros_w = jnp.zeros((n_pad, d), jnp.float32)
    ones_w = jnp.ones((K, d), jnp.float32)
    x_pad = jnp.concatenate([x, jnp.zeros((n_pad - n, d), x.dtype)])

    degp = _make_sc_degree(n_pad, cpt, d)(dst2d, zeros_w, ones_w)
    ct = NW * cpt
    q_total = ct // NS
    q0 = max(8, min(q_total - 8, int(round(0.85 * q_total / 8)) * 8))
    agg = _make_sc_agg(n_pad, ct, d, q0)

    y1 = _tc_first(x_pad, W1, degp, n)
    a1 = agg(y1, src2d, dst2d, zeros_w)
    y2 = _tc_mid(a1, y1, degp, b1, g1, be1, W2, n)
    a2 = agg(y2, src2d, dst2d, zeros_w)
    z = _tc_mid_nodot(a2, y2, degp, b2, g2, be2, n)
    a3 = agg(z, src2d, dst2d, zeros_w)
    out = _tc_last(a3, z, degp, W3, b3, n)
    return out[:n]


# speedup vs baseline: 1.4921x; 1.1230x over previous
import functools

import jax
import jax.numpy as jnp
from jax import lax
from jax.experimental import pallas as pl
from jax.experimental.pallas import tpu as pltpu
from jax.experimental.pallas import tpu_sc as plsc

NC = 2
NS = 16
NW = NC * NS
K = 128


def _sc_mesh():
    return plsc.VectorSubcoreMesh(core_axis_name="c", subcore_axis_name="s",
                                  num_cores=NC, num_subcores=NS)


@functools.lru_cache(maxsize=None)
def _make_sc_degree(n_pad, cpt, width):
    rpt = n_pad // NS

    @functools.partial(
        pl.kernel,
        out_type=jax.ShapeDtypeStruct((NC, n_pad, width), jnp.float32),
        mesh=_sc_mesh(),
        scratch_types=[
            pltpu.VMEM((cpt, K), jnp.int32),
            pltpu.VMEM((K, width), jnp.float32),
            pltpu.VMEM_SHARED((n_pad, width), jnp.float32),
        ],
    )
    def deg_kernel(dst_hbm, zeros_hbm, ones_hbm, out_hbm, idx_v, ones_v, acc):
        c = lax.axis_index("c")
        s = lax.axis_index("s")
        wid = c * NS + s
        pltpu.sync_copy(zeros_hbm.at[pl.ds(s * rpt, rpt)],
                        acc.at[pl.ds(s * rpt, rpt)])
        pltpu.sync_copy(ones_hbm, ones_v)
        pltpu.sync_copy(dst_hbm.at[pl.ds(wid * cpt, cpt)], idx_v)
        plsc.subcore_barrier()

        def body(j, carry):
            pltpu.sync_copy(ones_v, acc.at[idx_v.at[j]], add=True)
            return carry

        lax.fori_loop(0, cpt, body, 0)
        plsc.subcore_barrier()
        pltpu.sync_copy(acc.at[pl.ds(s * rpt, rpt)],
                        out_hbm.at[c, pl.ds(s * rpt, rpt)])

    return deg_kernel


@functools.lru_cache(maxsize=None)
def _make_sc_agg(n_pad, ct, width, q0):
    rpt = n_pad // NS
    q1 = (ct - NS * q0) // NS
    PCMAX = 48

    def phase_list(q):
        ph = [PCMAX] * (q // PCMAX)
        if q % PCMAX:
            ph.append(q % PCMAX)
        return ph

    @functools.partial(
        pl.kernel,
        out_type=jax.ShapeDtypeStruct((NC, n_pad, width), jnp.float32),
        mesh=_sc_mesh(),
        scratch_types=[
            pltpu.VMEM((PCMAX, K), jnp.int32),
            pltpu.VMEM((PCMAX, K), jnp.int32),
            pltpu.VMEM((K, width), jnp.float32),
            pltpu.VMEM((K, width), jnp.float32),
            pltpu.VMEM_SHARED((n_pad, width), jnp.float32),
            pltpu.SemaphoreType.DMA,
        ],
    )
    def agg_kernel(y_hbm, src_hbm, dst_hbm, zeros_hbm, out_hbm,
                   src_v, dst_v, rows0, rows1, acc, sem):
        c = lax.axis_index("c")
        s = lax.axis_index("s")
        pltpu.sync_copy(zeros_hbm.at[pl.ds(s * rpt, rpt)],
                        acc.at[pl.ds(s * rpt, rpt)])
        plsc.subcore_barrier()

        def run_phases(base, phases):
            off = 0
            for PC in phases:
                pltpu.sync_copy(src_hbm.at[pl.ds(base + off, PC)],
                                src_v.at[pl.ds(0, PC)])
                pltpu.sync_copy(dst_hbm.at[pl.ds(base + off, PC)],
                                dst_v.at[pl.ds(0, PC)])
                off += PC

                def gather(j, buf):
                    pltpu.async_copy(
                        y_hbm.at[src_v.at[j, pl.ds(0, K // 2)]],
                        buf.at[pl.ds(0, K // 2)], sem)
                    pltpu.async_copy(
                        y_hbm.at[src_v.at[j, pl.ds(K // 2, K // 2)]],
                        buf.at[pl.ds(K // 2, K // 2)], sem)

                def gather_wait(j, buf):
                    pltpu.make_async_copy(
                        y_hbm.at[src_v.at[j, pl.ds(0, K // 2)]],
                        buf.at[pl.ds(0, K // 2)], sem).wait()
                    pltpu.make_async_copy(
                        y_hbm.at[src_v.at[j, pl.ds(0, K // 2)]],
                        buf.at[pl.ds(K // 2, K // 2)], sem).wait()

                gather(0, rows0)

                def body(i, carry):
                    j0 = 2 * i
                    gather_wait(j0, rows0)
                    gather(j0 + 1, rows1)
                    pltpu.sync_copy(rows0, acc.at[dst_v.at[j0]], add=True)
                    gather_wait(j0 + 1, rows1)

                    @pl.when(j0 + 2 < PC)
                    def _():
                        gather(j0 + 2, rows0)

                    pltpu.sync_copy(rows1, acc.at[dst_v.at[j0 + 1]], add=True)
                    return carry

                lax.fori_loop(0, PC // 2, body, 0)

        @pl.when(c == 0)
        def _():
            run_phases(s * q0, phase_list(q0))

        if q1:
            @pl.when(c == 1)
            def _():
                run_phases(NS * q0 + s * q1, phase_list(q1))

        plsc.subcore_barrier()
        pltpu.sync_copy(acc.at[pl.ds(s * rpt, rpt)],
                        out_hbm.at[c, pl.ds(s * rpt, rpt)])

    return agg_kernel


def _dinv_col(degp_ref, n, n_pad):
    deg = degp_ref[0, :, 0:1] + degp_ref[1, :, 0:1] + 1.0
    mask = lax.broadcasted_iota(jnp.int32, (n_pad, 1), 0) < n
    return jnp.where(mask, lax.rsqrt(deg), 0.0), mask


def _tc_first(x_pad, w, degp, n):
    n_pad = x_pad.shape[0]

    def body(degp_ref, x_ref, w_ref, o_ref):
        dinv, _ = _dinv_col(degp_ref, n, n_pad)
        o_ref[...] = dinv * jnp.dot(x_ref[...], w_ref[...],
                                    preferred_element_type=jnp.float32)

    return pl.pallas_call(
        body,
        out_shape=jax.ShapeDtypeStruct((n_pad, w.shape[1]), jnp.float32),
    )(degp, x_pad, w)


def _bn_relu(aggp_ref, y_ref, degp_ref, b_ref, g_ref, be_ref, n, n_pad):
    dinv, mask = _dinv_col(degp_ref, n, n_pad)
    h = dinv * (aggp_ref[0] + aggp_ref[1] + y_ref[...]) + b_ref[...]
    h = jnp.where(mask, h, 0.0)
    m = jnp.sum(h, axis=0, keepdims=True) * (1.0 / n)
    ctr = jnp.where(mask, h - m, 0.0)
    v = jnp.sum(ctr * ctr, axis=0, keepdims=True) * (1.0 / n)
    hn = g_ref[...] * ctr * lax.rsqrt(v + 1e-5) + be_ref[...]
    hr = jnp.maximum(hn, 0.0)
    return jnp.where(mask, hr, 0.0), dinv


def _tc_mid(aggp, y, degp, b, g, be, w, n):
    n_pad = y.shape[0]

    def body(aggp_ref, y_ref, degp_ref, b_ref, g_ref, be_ref, w_ref, o_ref):
        hr, dinv = _bn_relu(aggp_ref, y_ref, degp_ref, b_ref, g_ref, be_ref,
                            n, n_pad)
        o_ref[...] = dinv * jnp.dot(hr, w_ref[...],
                                    preferred_element_type=jnp.float32)

    return pl.pallas_call(
        body,
        out_shape=jax.ShapeDtypeStruct((n_pad, w.shape[1]), jnp.float32),
    )(aggp, y, degp, b, g, be, w)


def _tc_mid_nodot(aggp, y, degp, b, g, be, n):
    n_pad = y.shape[0]

    def body(aggp_ref, y_ref, degp_ref, b_ref, g_ref, be_ref, o_ref):
        hr, dinv = _bn_relu(aggp_ref, y_ref, degp_ref, b_ref, g_ref, be_ref,
                            n, n_pad)
        o_ref[...] = dinv * hr

    return pl.pallas_call(
        body,
        out_shape=jax.ShapeDtypeStruct((n_pad, y.shape[1]), jnp.float32),
    )(aggp, y, degp, b, g, be)


def _tc_last(aggp, z, degp, w3, b3, n):
    n_pad = z.shape[0]

    def body(aggp_ref, z_ref, degp_ref, w3_ref, b3_ref, o_ref):
        dinv, _ = _dinv_col(degp_ref, n, n_pad)
        t = dinv * (aggp_ref[0] + aggp_ref[1] + z_ref[...])
        logits = jnp.dot(t, w3_ref[...],
                         preferred_element_type=jnp.float32) + b3_ref[...]
        mx = jnp.max(logits, axis=-1, keepdims=True)
        sh = logits - mx
        lse = jnp.log(jnp.sum(jnp.exp(sh), axis=-1, keepdims=True))
        o_ref[...] = sh - lse

    return pl.pallas_call(
        body,
        out_shape=jax.ShapeDtypeStruct((n_pad, w3.shape[1]), jnp.float32),
    )(aggp, z, degp, w3, b3)


def kernel(x, adj_t, W1, b1, g1, be1, W2, b2, g2, be2, W3, b3):
    n, d = x.shape
    e = adj_t.shape[1]
    cpt = 8 * (-(-e // (NW * K * 8)))
    e_pad = NW * K * cpt
    rpt = 8 * (-(-(n + 1) // (NS * 8)))
    n_pad = NS * rpt

    pad = jnp.full((e_pad - e,), n, dtype=jnp.int32)
    src2d = jnp.concatenate([adj_t[0], pad]).reshape(NW * cpt, K)
    dst2d = jnp.concatenate([adj_t[1], pad]).reshape(NW * cpt, K)
    zeros_w = jnp.zeros((n_pad, d), jnp.float32)
    ones_w = jnp.ones((K, d), jnp.float32)
    x_pad = jnp.concatenate([x, jnp.zeros((n_pad - n, d), x.dtype)])

    degp = _make_sc_degree(n_pad, cpt, d)(dst2d, zeros_w, ones_w)
    ct = NW * cpt
    q_total = ct // NS
    q0 = max(8, min(q_total - 8, int(round(0.95 * q_total / 8)) * 8))
    agg = _make_sc_agg(n_pad, ct, d, q0)

    y1 = _tc_first(x_pad, W1, degp, n)
    a1 = agg(y1, src2d, dst2d, zeros_w)
    y2 = _tc_mid(a1, y1, degp, b1, g1, be1, W2, n)
    a2 = agg(y2, src2d, dst2d, zeros_w)
    z = _tc_mid_nodot(a2, y2, degp, b2, g2, be2, n)
    a3 = agg(z, src2d, dst2d, zeros_w)
    out = _tc_last(a3, z, degp, W3, b3, n)
    return out[:n]
